# baseline (speedup 1.0000x reference)
import functools

import jax
import jax.numpy as jnp
from jax import lax
from jax.experimental import pallas as pl
from jax.experimental.pallas import tpu as pltpu
from jax.experimental.pallas import tpu_sc as plsc

QB = 256


def _kcols_body(x_ref, wkt_ref, kc_ref):
    kc_ref[0] = jnp.dot(x_ref[0], wkt_ref[...], preferred_element_type=jnp.float32)


def _main_body(x_ref, wq_ref, kc_ref, wvt_ref, v_ref, idx_ref):
    xb = x_ref[...]
    v_ref[...] = jnp.dot(xb, wvt_ref[...], preferred_element_type=jnp.float32)
    qt = lax.dot_general(wq_ref[...], xb, (((1,), (1,)), ((), ())),
                         preferred_element_type=jnp.float32)
    kc = kc_ref[...]
    t = kc.shape[0]
    s = lax.dot_general(kc.astype(jnp.bfloat16), qt.astype(jnp.bfloat16),
                        (((1,), (0,)), ((), ())),
                        preferred_element_type=jnp.float32)
    m = jnp.max(s, axis=0, keepdims=True)
    iota = lax.broadcasted_iota(jnp.int32, s.shape, 0)
    cand = jnp.where(s == m, iota, t)
    idx_ref[0, 0, :] = jnp.min(cand, axis=0)


def _sc_gather(table, gidx):
    rows, d = table.shape
    nw = 32
    b_per_w = rows // nw
    ch = 64
    nch = b_per_w // ch
    mesh = plsc.VectorSubcoreMesh(core_axis_name="c", subcore_axis_name="s")

    @functools.partial(
        pl.kernel,
        mesh=mesh,
        out_type=jax.ShapeDtypeStruct((rows, d), jnp.float32),
        scratch_types=[
            pltpu.VMEM((ch,), jnp.int32),
            pltpu.VMEM((ch, d), jnp.float32),
            pltpu.VMEM((ch,), jnp.int32),
            pltpu.VMEM((ch, d), jnp.float32),
            pltpu.SemaphoreType.DMA,
            pltpu.SemaphoreType.DMA,
            pltpu.SemaphoreType.DMA,
            pltpu.SemaphoreType.DMA,
        ],
    )
    def k(table_hbm, idx_hbm, out_hbm, idx_a, rows_a, idx_b, rows_b,
          sg_a, sg_b, sw_a, sw_b):
        wid = lax.axis_index("s") * 2 + lax.axis_index("c")
        base = wid * b_per_w
        ib = [idx_a, idx_b]
        rb = [rows_a, rows_b]
        sg = [sg_a, sg_b]
        sw = [sw_a, sw_b]
        g = [None, None]
        w = [None, None]

        def issue_gather(c):
            j = c % 2
            pltpu.sync_copy(idx_hbm.at[pl.ds(base + c * ch, ch)], ib[j])
            g[j] = pltpu.async_copy(table_hbm.at[ib[j]], rb[j], sg[j])

        issue_gather(0)
        for c in range(nch):
            j = c % 2
            if c + 1 < nch:
                if c >= 1:
                    w[(c + 1) % 2].wait()
                issue_gather(c + 1)
            g[j].wait()
            w[j] = pltpu.async_copy(rb[j], out_hbm.at[pl.ds(base + c * ch, ch)],
                                    sw[j])
        for c in range(max(0, nch - 2), nch):
            w[c % 2].wait()

    return k(table, gidx)


def kernel(x, W_Q, W_K, W_V):
    B, T, D = x.shape
    nqb = T // QB
    wkt8 = jnp.pad(W_K.T, ((0, 0), (0, 8 - W_K.shape[0])))
    wq8 = jnp.pad(W_Q, ((0, 8 - W_Q.shape[0]), (0, 0)))
    wvt = W_V.T

    kcols = pl.pallas_call(
        _kcols_body,
        grid=(B, nqb),
        in_specs=[
            pl.BlockSpec((1, QB, D), lambda b, t: (b, t, 0)),
            pl.BlockSpec((D, 8), lambda b, t: (0, 0)),
        ],
        out_specs=pl.BlockSpec((1, QB, 8), lambda b, t: (b, t, 0)),
        out_shape=jax.ShapeDtypeStruct((B, T, 8), jnp.float32),
    )(x, wkt8)

    main_call = pl.pallas_call(
        _main_body,
        grid=(nqb,),
        in_specs=[
            pl.BlockSpec((QB, D), lambda t: (t, 0)),
            pl.BlockSpec((8, D), lambda t: (0, 0)),
            pl.BlockSpec((T, 8), lambda t: (0, 0)),
            pl.BlockSpec((D, D), lambda t: (0, 0)),
        ],
        out_specs=[
            pl.BlockSpec((QB, D), lambda t: (t, 0)),
            pl.BlockSpec((1, 1, QB), lambda t: (t, 0, 0)),
        ],
        out_shape=[
            jax.ShapeDtypeStruct((T, D), jnp.float32),
            jax.ShapeDtypeStruct((nqb, 1, QB), jnp.int32),
        ],
    )

    outs = []
    for b in range(B):
        v_b, idx_b = main_call(x[b], wq8, kcols[b], wvt)
        outs.append(_sc_gather(v_b, idx_b.reshape(T)))
    return jnp.stack(outs)


# speedup vs baseline: 1.3719x; 1.3719x over previous
import functools

import jax
import jax.numpy as jnp
from jax import lax
from jax.experimental import pallas as pl
from jax.experimental.pallas import tpu as pltpu
from jax.experimental.pallas import tpu_sc as plsc

QB = 256
KB = 1024


def _tree_max(x):
    while x.shape[0] > 8:
        h = x.shape[0] // 2
        x = jnp.maximum(x[:h], x[h:])
    return jnp.max(x, axis=0, keepdims=True)


def _tree_min(x):
    while x.shape[0] > 8:
        h = x.shape[0] // 2
        x = jnp.minimum(x[:h], x[h:])
    return jnp.min(x, axis=0, keepdims=True)


def _kcols_body(x_ref, wkt_ref, kc_ref):
    kc_ref[0] = jnp.dot(x_ref[0], wkt_ref[...], preferred_element_type=jnp.float32)


def _main_body(x_ref, wq_ref, kc_ref, wvt_ref, v_ref, idx_ref):
    xb = x_ref[0]
    v_ref[0] = jnp.dot(xb, wvt_ref[...], preferred_element_type=jnp.float32)
    qt = lax.dot_general(wq_ref[...], xb, (((1,), (1,)), ((), ())),
                         preferred_element_type=jnp.float32)
    kc = kc_ref[0]
    t = kc.shape[0]
    s = lax.dot_general(kc.astype(jnp.bfloat16), qt.astype(jnp.bfloat16),
                        (((1,), (0,)), ((), ())),
                        preferred_element_type=jnp.float32)
    m = _tree_max(s)
    iota = lax.broadcasted_iota(jnp.int32, s.shape, 0)
    cand = jnp.where(s == m, iota, t)
    idx = _tree_min(cand)[0]
    b = pl.program_id(0)
    idx_ref[0, 0, :] = idx + b * t


def _sc_gather(table, gidx):
    rows, d = table.shape
    nw = 32
    b_per_w = rows // nw
    ch = 64
    nch = b_per_w // ch
    mesh = plsc.VectorSubcoreMesh(core_axis_name="c", subcore_axis_name="s")

    @functools.partial(
        pl.kernel,
        mesh=mesh,
        out_type=jax.ShapeDtypeStruct((rows, d), jnp.float32),
        scratch_types=[
            pltpu.VMEM((ch,), jnp.int32),
            pltpu.VMEM((ch, d), jnp.float32),
            pltpu.VMEM((ch,), jnp.int32),
            pltpu.VMEM((ch, d), jnp.float32),
            pltpu.SemaphoreType.DMA,
            pltpu.SemaphoreType.DMA,
        ],
    )
    def k(table_hbm, idx_hbm, out_hbm, idx_a, rows_a, idx_b, rows_b, sem_a, sem_b):
        wid = lax.axis_index("s") * 2 + lax.axis_index("c")
        base = wid * b_per_w
        bufs = [(idx_a, rows_a, sem_a), (idx_b, rows_b, sem_b)]
        copies = [None, None]

        def issue(c):
            ib, rb, sm = bufs[c % 2]
            pltpu.sync_copy(idx_hbm.at[pl.ds(base + c * ch, ch)], ib)
            copies[c % 2] = pltpu.async_copy(table_hbm.at[ib], rb, sm)

        issue(0)
        for c in range(nch):
            if c + 1 < nch:
                issue(c + 1)
            _, rb, _ = bufs[c % 2]
            copies[c % 2].wait()
            pltpu.sync_copy(rb, out_hbm.at[pl.ds(base + c * ch, ch)])

    return k(table, gidx)


def kernel(x, W_Q, W_K, W_V):
    B, T, D = x.shape
    nqb = T // QB
    wkt8 = jnp.pad(W_K.T, ((0, 0), (0, 8 - W_K.shape[0])))
    wq8 = jnp.pad(W_Q, ((0, 8 - W_Q.shape[0]), (0, 0)))
    wvt = W_V.T

    kcols = pl.pallas_call(
        _kcols_body,
        grid=(B, T // KB),
        in_specs=[
            pl.BlockSpec((1, KB, D), lambda b, t: (b, t, 0)),
            pl.BlockSpec((D, 8), lambda b, t: (0, 0)),
        ],
        out_specs=pl.BlockSpec((1, KB, 8), lambda b, t: (b, t, 0)),
        out_shape=jax.ShapeDtypeStruct((B, T, 8), jnp.float32),
    )(x, wkt8)

    v, idx = pl.pallas_call(
        _main_body,
        grid=(B, nqb),
        in_specs=[
            pl.BlockSpec((1, QB, D), lambda b, t: (b, t, 0)),
            pl.BlockSpec((8, D), lambda b, t: (0, 0)),
            pl.BlockSpec((1, T, 8), lambda b, t: (b, 0, 0)),
            pl.BlockSpec((D, D), lambda b, t: (0, 0)),
        ],
        out_specs=[
            pl.BlockSpec((1, QB, D), lambda b, t: (b, t, 0)),
            pl.BlockSpec((1, 1, QB), lambda b, t: (b * nqb + t, 0, 0)),
        ],
        out_shape=[
            jax.ShapeDtypeStruct((B, T, D), jnp.float32),
            jax.ShapeDtypeStruct((B * nqb, 1, QB), jnp.int32),
        ],
    )(x, wq8, kcols, wvt)

    table = v.reshape(B * T, D)
    gidx = idx.reshape(B * T)
    out = _sc_gather(table, gidx)
    return out.reshape(B, T, D)
